# Initial kernel scaffold; baseline (speedup 1.0000x reference)
import jax
import jax.numpy as jnp
from jax.experimental import pallas as pl


def kernel(pred_H_sampled, pred_batch_ix, true_index_sampled, true_H_sampled):
    raise NotImplementedError("write your pallas kernel here")


# speedup vs baseline: 1.3730x; 1.3730x over previous
import functools

import jax
import jax.numpy as jnp
from jax import lax
from jax.experimental import pallas as pl
from jax.experimental.pallas import tpu as pltpu
from jax.experimental.pallas import tpu_sc as plsc

_B = 16
_N = 100000
_S = 50000
_HALF = _S // 2
_CHUNK = 8336
_STARTS = (0, _CHUNK, _HALF - _CHUNK)
_VALID_FROM = (0, 0, 2 * _CHUNK - _HALF)
_STEPS = _CHUNK // 16


def _body(pred_h, bix, tidx, th, out, row_v, idx_v, th_v, bix_v, row_ix_v, acc_v):
    c = lax.axis_index("c")
    s = lax.axis_index("s")
    lanes = lax.iota(jnp.int32, 16)

    pltpu.sync_copy(bix, bix_v)
    svec = jnp.zeros((16,), jnp.int32) + s
    rvec = lax.gather(
        bix_v[...],
        svec[:, None],
        dimension_numbers=lax.GatherDimensionNumbers(
            offset_dims=(), collapsed_slice_dims=(0,), start_index_map=(0,)
        ),
        slice_sizes=(1,),
        mode=lax.GatherScatterMode.PROMISE_IN_BOUNDS,
    )
    row_ix_v[...] = rvec
    pltpu.sync_copy(pred_h.at[row_ix_v.at[pl.ds(0, 1)]], row_v)

    base = s * _S + c * _HALF
    zeros16 = jnp.zeros((16,), jnp.int32)
    acc = jnp.zeros((16,), jnp.float32)
    for ci in range(3):
        st = pl.multiple_of(base + _STARTS[ci], 8)
        pltpu.sync_copy(tidx.at[pl.ds(st, _CHUNK)], idx_v)
        pltpu.sync_copy(th.at[pl.ds(st, _CHUNK)], th_v)
        vf = _VALID_FROM[ci]

        def step(j, a, _vf=vf):
            off = j * 16
            iv = idx_v[pl.ds(off, 16)]
            g = plsc.load_gather(row_v.at[0], [iv])
            t = th_v[pl.ds(off, 16)]
            d = g - t
            if _vf:
                m = (lanes + off) >= _vf
                return a + jnp.where(m, d * d, 0.0)
            return a + d * d

        acc = lax.fori_loop(0, _STEPS, step, acc)

    acc_v[...] = acc
    wid = s * 2 + c
    pltpu.sync_copy(acc_v, out.at[pl.ds(pl.multiple_of(wid * 16, 8), 16)])


@jax.jit
def _sc_partials(pred_h, bix_i32, tidx_i32, th):
    mesh = plsc.VectorSubcoreMesh(core_axis_name="c", subcore_axis_name="s")
    return pl.kernel(
        _body,
        out_type=jax.ShapeDtypeStruct((_B * 2 * 16,), jnp.float32),
        mesh=mesh,
        compiler_params=pltpu.CompilerParams(
            needs_layout_passes=False, use_tc_tiling_on_sc=False
        ),
        scratch_types=[
            pltpu.VMEM((1, _N), jnp.float32),
            pltpu.VMEM((_CHUNK,), jnp.int32),
            pltpu.VMEM((_CHUNK,), jnp.float32),
            pltpu.VMEM((16,), jnp.int32),
            pltpu.VMEM((16,), jnp.int32),
            pltpu.VMEM((16,), jnp.float32),
        ],
    )(pred_h, bix_i32, tidx_i32, th)


def kernel(pred_H_sampled, pred_batch_ix, true_index_sampled, true_H_sampled):
    bix = pred_batch_ix.astype(jnp.int32)
    tidx = true_index_sampled.astype(jnp.int32).reshape(-1)
    partials = _sc_partials(
        pred_H_sampled, bix, tidx, true_H_sampled.reshape(-1)
    )
    return jnp.sum(partials) / (_B * _S)


# speedup vs baseline: 1.6665x; 1.2138x over previous
import jax
import jax.numpy as jnp
from jax import lax
from jax.experimental import pallas as pl
from jax.experimental.pallas import tpu as pltpu
from jax.experimental.pallas import tpu_sc as plsc

_B = 16
_N = 100000
_S = 50000
_HALF = _S // 2
_CHUNK = 6272
_UNROLL = 8
_OUTER = _CHUNK // (16 * _UNROLL)
_STARTS = (0, _CHUNK, 2 * _CHUNK, _HALF - _CHUNK)
_VALID_FROM = (0, 0, 0, 4 * _CHUNK - _HALF)


def _body(pred_h, bix, tidx, th, out,
          row_v, idx_v0, th_v0, idx_v1, th_v1, bix_v, row_ix_v, acc_v,
          sem_row, sem_i0, sem_t0, sem_i1, sem_t1):
    c = lax.axis_index("c")
    s = lax.axis_index("s")
    lanes = lax.iota(jnp.int32, 16)

    pltpu.sync_copy(bix, bix_v)
    svec = jnp.zeros((16,), jnp.int32) + s
    rvec = lax.gather(
        bix_v[...],
        svec[:, None],
        dimension_numbers=lax.GatherDimensionNumbers(
            offset_dims=(), collapsed_slice_dims=(0,), start_index_map=(0,)
        ),
        slice_sizes=(1,),
        mode=lax.GatherScatterMode.PROMISE_IN_BOUNDS,
    )
    row_ix_v[...] = rvec
    row_cp = pltpu.async_copy(pred_h.at[row_ix_v.at[pl.ds(0, 1)]], row_v, sem_row)

    base = s * _S + c * _HALF
    bufs = ((idx_v0, th_v0, sem_i0, sem_t0), (idx_v1, th_v1, sem_i1, sem_t1))

    def issue(ci):
        st = pl.multiple_of(base + _STARTS[ci], 8)
        iv, tv, si, st_sem = bufs[ci % 2]
        cp_i = pltpu.async_copy(tidx.at[pl.ds(st, _CHUNK)], iv, si)
        cp_t = pltpu.async_copy(th.at[pl.ds(st, _CHUNK)], tv, st_sem)
        return cp_i, cp_t

    inflight = [issue(0), issue(1)]
    row_cp.wait()

    accs = [jnp.zeros((16,), jnp.float32) for _ in range(_UNROLL)]
    for ci in range(4):
        cp_i, cp_t = inflight[ci % 2]
        cp_i.wait()
        cp_t.wait()
        idx_ref, th_ref = bufs[ci % 2][0], bufs[ci % 2][1]
        vf = _VALID_FROM[ci]

        def one(off, a, m, _idx=idx_ref, _th=th_ref):
            iv = _idx[pl.ds(off, 16)]
            g = plsc.load_gather(row_v.at[0], [iv])
            t = _th[pl.ds(off, 16)]
            d = g - t
            sq = d * d
            if m is not None:
                sq = sq * m
            return a + sq

        def step(j, accs_t):
            return tuple(
                one((j * _UNROLL + u) * 16, accs_t[u], None)
                for u in range(_UNROLL)
            )

        if vf:
            for u in range(_UNROLL):
                off = u * 16
                if off + 16 <= vf:
                    continue
                m = None
                if off < vf:
                    m = (lanes >= (vf - off)).astype(jnp.float32)
                accs[u] = one(off, accs[u], m)
            accs = list(lax.fori_loop(1, _OUTER, step, tuple(accs)))
        else:
            accs = list(lax.fori_loop(0, _OUTER, step, tuple(accs)))
        if ci + 2 < 4:
            inflight[ci % 2] = issue(ci + 2)

    total = accs[0]
    for u in range(1, _UNROLL):
        total = total + accs[u]
    acc_v[...] = total
    wid = s * 2 + c
    pltpu.sync_copy(acc_v, out.at[pl.ds(pl.multiple_of(wid * 16, 8), 16)])


@jax.jit
def _sc_partials(pred_h, bix_i32, tidx_i32, th):
    mesh = plsc.VectorSubcoreMesh(core_axis_name="c", subcore_axis_name="s")
    return pl.kernel(
        _body,
        out_type=jax.ShapeDtypeStruct((_B * 2 * 16,), jnp.float32),
        mesh=mesh,
        compiler_params=pltpu.CompilerParams(
            needs_layout_passes=False, use_tc_tiling_on_sc=False
        ),
        scratch_types=[
            pltpu.VMEM((1, _N), jnp.float32),
            pltpu.VMEM((_CHUNK,), jnp.int32),
            pltpu.VMEM((_CHUNK,), jnp.float32),
            pltpu.VMEM((_CHUNK,), jnp.int32),
            pltpu.VMEM((_CHUNK,), jnp.float32),
            pltpu.VMEM((16,), jnp.int32),
            pltpu.VMEM((16,), jnp.int32),
            pltpu.VMEM((16,), jnp.float32),
            pltpu.SemaphoreType.DMA,
            pltpu.SemaphoreType.DMA,
            pltpu.SemaphoreType.DMA,
            pltpu.SemaphoreType.DMA,
            pltpu.SemaphoreType.DMA,
        ],
    )(pred_h, bix_i32, tidx_i32, th)


def kernel(pred_H_sampled, pred_batch_ix, true_index_sampled, true_H_sampled):
    bix = pred_batch_ix.astype(jnp.int32)
    tidx = true_index_sampled.astype(jnp.int32).reshape(-1)
    partials = _sc_partials(
        pred_H_sampled, bix, tidx, true_H_sampled.reshape(-1)
    )
    return jnp.sum(partials) / (_B * _S)


# speedup vs baseline: 1.8460x; 1.1077x over previous
import jax
import jax.numpy as jnp
from jax import lax
from jax.experimental import pallas as pl
from jax.experimental.pallas import tpu as pltpu
from jax.experimental.pallas import tpu_sc as plsc

_B = 16
_N = 100000
_S = 50000
_HALF = _S // 2
_CHUNK = 6272
_UNROLL = 8
_OUTER = _CHUNK // (16 * _UNROLL)
_STARTS = (0, _CHUNK, 2 * _CHUNK, _HALF - _CHUNK)
_VALID_FROM = (0, 0, 0, 4 * _CHUNK - _HALF)
_RSTRIDE = 100352
_CSTRIDE = 50176


def _prep_body(bix_ref, ph_ref, tidx_ref, th_ref, rows_ref, tidxo_ref, tho_ref):
    r = pl.program_id(1)
    i = pl.program_id(0) * 8 + r
    rows_ref[pl.ds(0, _N)] = ph_ref[bix_ref[i]]
    tidxo_ref[pl.ds(0, _S)] = tidx_ref[r]
    tho_ref[pl.ds(0, _S)] = th_ref[r]


@jax.jit
def _prep(pred_h, bix_i32, tidx, th):
    grid_spec = pltpu.PrefetchScalarGridSpec(
        num_scalar_prefetch=1,
        grid=(2, 8),
        in_specs=[
            pl.BlockSpec((_B, _N), lambda g, r, bix: (0, 0)),
            pl.BlockSpec((8, _S), lambda g, r, bix: (g, 0)),
            pl.BlockSpec((8, _S), lambda g, r, bix: (g, 0)),
        ],
        out_specs=[
            pl.BlockSpec((_RSTRIDE,), lambda g, r, bix: (g * 8 + r,)),
            pl.BlockSpec((_CSTRIDE,), lambda g, r, bix: (g * 8 + r,)),
            pl.BlockSpec((_CSTRIDE,), lambda g, r, bix: (g * 8 + r,)),
        ],
    )
    return pl.pallas_call(
        _prep_body,
        grid_spec=grid_spec,
        out_shape=[
            jax.ShapeDtypeStruct((_B * _RSTRIDE,), jnp.float32),
            jax.ShapeDtypeStruct((_B * _CSTRIDE,), jnp.int32),
            jax.ShapeDtypeStruct((_B * _CSTRIDE,), jnp.float32),
        ],
    )(bix_i32, pred_h, tidx, th)


def _body(rows, tidx, th, out,
          row_v, idx_v0, th_v0, idx_v1, th_v1, acc_v,
          sem_row, sem_i0, sem_t0, sem_i1, sem_t1):
    c = lax.axis_index("c")
    s = lax.axis_index("s")
    lanes = lax.iota(jnp.int32, 16)

    row_cp = pltpu.async_copy(
        rows.at[pl.ds(pl.multiple_of(s * _RSTRIDE, 8), _N)], row_v, sem_row
    )

    base = s * _CSTRIDE + c * _HALF
    bufs = ((idx_v0, th_v0, sem_i0, sem_t0), (idx_v1, th_v1, sem_i1, sem_t1))

    def issue(ci):
        st = pl.multiple_of(base + _STARTS[ci], 8)
        iv, tv, si, st_sem = bufs[ci % 2]
        cp_i = pltpu.async_copy(tidx.at[pl.ds(st, _CHUNK)], iv, si)
        cp_t = pltpu.async_copy(th.at[pl.ds(st, _CHUNK)], tv, st_sem)
        return cp_i, cp_t

    inflight = [issue(0), issue(1)]
    row_cp.wait()

    accs = [jnp.zeros((16,), jnp.float32) for _ in range(_UNROLL)]
    for ci in range(4):
        cp_i, cp_t = inflight[ci % 2]
        cp_i.wait()
        cp_t.wait()
        idx_ref, th_ref = bufs[ci % 2][0], bufs[ci % 2][1]
        vf = _VALID_FROM[ci]

        def one(off, a, m, _idx=idx_ref, _th=th_ref):
            iv = _idx[pl.ds(off, 16)]
            g = plsc.load_gather(row_v, [iv])
            t = _th[pl.ds(off, 16)]
            d = g - t
            sq = d * d
            if m is not None:
                sq = sq * m
            return a + sq

        def step(j, accs_t):
            return tuple(
                one((j * _UNROLL + u) * 16, accs_t[u], None)
                for u in range(_UNROLL)
            )

        if vf:
            for u in range(_UNROLL):
                off = u * 16
                if off + 16 <= vf:
                    continue
                m = None
                if off < vf:
                    m = (lanes >= (vf - off)).astype(jnp.float32)
                accs[u] = one(off, accs[u], m)
            accs = list(lax.fori_loop(1, _OUTER, step, tuple(accs)))
        else:
            accs = list(lax.fori_loop(0, _OUTER, step, tuple(accs)))

        if ci + 2 < 4:
            inflight[ci % 2] = issue(ci + 2)

    total = accs[0]
    for u in range(1, _UNROLL):
        total = total + accs[u]
    acc_v[...] = total
    wid = s * 2 + c
    pltpu.sync_copy(acc_v, out.at[pl.ds(pl.multiple_of(wid * 16, 8), 16)])


@jax.jit
def _sc_partials(rows, tidx_lin, th_lin):
    mesh = plsc.VectorSubcoreMesh(core_axis_name="c", subcore_axis_name="s")
    return pl.kernel(
        _body,
        out_type=jax.ShapeDtypeStruct((_B * 2 * 16,), jnp.float32),
        mesh=mesh,
        compiler_params=pltpu.CompilerParams(
            needs_layout_passes=False, use_tc_tiling_on_sc=False
        ),
        scratch_types=[
            pltpu.VMEM((_N,), jnp.float32),
            pltpu.VMEM((_CHUNK,), jnp.int32),
            pltpu.VMEM((_CHUNK,), jnp.float32),
            pltpu.VMEM((_CHUNK,), jnp.int32),
            pltpu.VMEM((_CHUNK,), jnp.float32),
            pltpu.VMEM((16,), jnp.float32),
            pltpu.SemaphoreType.DMA,
            pltpu.SemaphoreType.DMA,
            pltpu.SemaphoreType.DMA,
            pltpu.SemaphoreType.DMA,
            pltpu.SemaphoreType.DMA,
        ],
    )(rows, tidx_lin, th_lin)


def kernel(pred_H_sampled, pred_batch_ix, true_index_sampled, true_H_sampled):
    bix = pred_batch_ix.astype(jnp.int32)
    tidx = true_index_sampled.astype(jnp.int32)
    rows, tidx_lin, th_lin = _prep(pred_H_sampled, bix, tidx, true_H_sampled)
    partials = _sc_partials(rows, tidx_lin, th_lin)
    return jnp.sum(partials) / (_B * _S)
